# baseline (speedup 1.0000x reference)
import jax
import jax.numpy as jnp
from jax.experimental import pallas as pl
from jax.experimental.pallas import tpu as pltpu

_B = 64
_GRID = 8
_EMB = 256
_DIM = 1024
_K = 8192
_CH = 32
_NE = 2 * _B // _CH
_KB = _K // _NE


def _enc_body(x, w_ref, b_ref, c1_ref, c1b_ref, c2_ref, c2b_ref, after_proj):
    y = jnp.dot(x, w_ref[...],
                preferred_element_type=jnp.float32) + b_ref[...]
    after_proj()
    y4 = y.reshape(_CH, _GRID, _GRID, _EMB)
    yp = jnp.pad(y4, ((0, 0), (1, 1), (1, 1), (0, 0)))
    y6 = yp.reshape(_CH, 5, 2, 5, 2, _EMB)
    acc = jnp.broadcast_to(c1b_ref[...], (_CH * 16, _EMB))
    for di in range(3):
        fi, pi = di // 2, di % 2
        for dj in range(3):
            fj, pj = dj // 2, dj % 2
            xs = y6[:, fi:fi + 4, pi, fj:fj + 4, pj, :].reshape(
                _CH * 16, _EMB)
            acc = acc + jnp.dot(xs, c1_ref[di * 3 + dj],
                                preferred_element_type=jnp.float32)
    h = jnp.maximum(acc, 0.0).reshape(_CH, 16 * _EMB)
    return jnp.dot(h, c2_ref[...],
                   preferred_element_type=jnp.float32) + c2b_ref[...]


def _fused_kernel(x1_ref, x2_ref, w_ref, b_ref, c1_ref, c1b_ref, c2_ref,
                  c2b_ref, cbh_ref, rv_ref, wout_ref, bout_ref,
                  out_ref, perp_ref, e_ref, cb_ref, x2_sc, c1_sc, c2_sc,
                  sem, xsem, wsem):
    i = pl.program_id(0)
    nh = _NE // 2

    @pl.when(i == 0)
    def _():
        pltpu.make_async_copy(c1_ref, c1_sc, wsem.at[0]).start()
        pltpu.make_async_copy(c2_ref, c2_sc, wsem.at[1]).start()

    def _wait_w():
        @pl.when(i == 0)
        def _():
            pltpu.make_async_copy(c1_ref, c1_sc, wsem.at[0]).wait()
            pltpu.make_async_copy(c2_ref, c2_sc, wsem.at[1]).wait()

    @pl.when(i < _NE)
    def _():
        pltpu.make_async_copy(
            cbh_ref.at[pl.ds(i * _KB, _KB), :],
            cb_ref.at[pl.ds(i * _KB, _KB), :], sem).start()

        @pl.when(i < nh)
        def _():
            pltpu.make_async_copy(
                x2_ref.at[pl.ds(i * _CH * 64, _CH * 64), :],
                x2_sc.at[i], xsem.at[i]).start()
            e_ref[pl.ds(i * _CH, _CH), :] = _enc_body(
                x1_ref[...], w_ref, b_ref, c1_sc, c1b_ref, c2_sc, c2b_ref,
                _wait_w)

        @pl.when(i >= nh)
        def _():
            j = i - nh
            pltpu.make_async_copy(
                x2_ref.at[pl.ds(j * _CH * 64, _CH * 64), :],
                x2_sc.at[j], xsem.at[j]).wait()
            e_ref[pl.ds(i * _CH, _CH), :] = _enc_body(
                x2_sc[j], w_ref, b_ref, c1_sc, c1b_ref, c2_sc, c2b_ref,
                _wait_w)

    @pl.when(i == _NE)
    def _():
        for d in range(_NE):
            pltpu.make_async_copy(
                cbh_ref.at[pl.ds(d * _KB, _KB), :],
                cb_ref.at[pl.ds(d * _KB, _KB), :], sem).wait()
        e = e_ref[...]
        z = e[_B:, :] - e[:_B, :]
        cb = cb_ref[...]
        ones = jnp.ones((1, _EMB), dtype=jnp.float32)
        cbn = jax.lax.dot_general(ones, cb * cb, (((1,), (1,)), ((), ())),
                                  preferred_element_type=jnp.float32)
        cross = jax.lax.dot_general(z, cb, (((1,), (1,)), ((), ())),
                                    preferred_element_type=jnp.float32)
        s = cbn - 2.0 * cross
        smin = jnp.min(s, axis=1, keepdims=True)
        iota = jax.lax.broadcasted_iota(jnp.int32, (_B, _K), 1)
        idx = jnp.min(jnp.where(s <= smin, iota, _K), axis=1, keepdims=True)
        onehot = (iota == idx).astype(jnp.float32)
        hard = jnp.dot(onehot, cb, preferred_element_type=jnp.float32)

        dz = z - hard
        nq = jnp.sqrt(jnp.sum(dz * dz, axis=1, keepdims=True))
        rv = rv_ref[...]
        nr = jnp.sqrt(jnp.sum(rv * rv, axis=1, keepdims=True))
        quantized = z + (nq / (nr + 1e-12)) * rv

        ieq = (idx == jnp.transpose(idx)).astype(jnp.float32)
        counts = jnp.sum(ieq, axis=1, keepdims=True)
        lp = jnp.log(counts * (1.0 / _B) + 1e-10)
        perp_ref[...] = jnp.broadcast_to(
            jnp.exp(-jnp.sum(lp) * (1.0 / _B)), (1, 1))

        out_ref[...] = (jnp.dot(quantized, wout_ref[...],
                                preferred_element_type=jnp.float32)
                        + bout_ref[...])


def kernel(input_data_first, input_data_last, codebooks, Win, b_in, Wout,
           b_out, c1w, c1b, c2w, c2b):
    c1r = jnp.transpose(c1w, (2, 3, 1, 0)).reshape(9, _EMB, _EMB)
    c2r = jnp.transpose(c2w, (2, 3, 1, 0)).reshape(16 * _EMB, _EMB)
    rv = jax.random.normal(jax.random.key(42), (_B, _EMB), dtype=jnp.float32)

    out, perp = pl.pallas_call(
        _fused_kernel,
        grid=(_NE + 1,),
        in_specs=[
            pl.BlockSpec((_CH * 64, _DIM), lambda i: (jnp.minimum(i, _NE // 2 - 1), 0)),
            pl.BlockSpec(memory_space=pltpu.MemorySpace.HBM),
            pl.BlockSpec((_DIM, _EMB), lambda i: (0, 0)),
            pl.BlockSpec((1, _EMB), lambda i: (0, 0)),
            pl.BlockSpec(memory_space=pltpu.MemorySpace.HBM),
            pl.BlockSpec((1, _EMB), lambda i: (0, 0)),
            pl.BlockSpec(memory_space=pltpu.MemorySpace.HBM),
            pl.BlockSpec((1, _EMB), lambda i: (0, 0)),
            pl.BlockSpec(memory_space=pltpu.MemorySpace.HBM),
            pl.BlockSpec((_B, _EMB), lambda i: (0, 0)),
            pl.BlockSpec((_EMB, _DIM), lambda i: (0, 0)),
            pl.BlockSpec((1, _DIM), lambda i: (0, 0)),
        ],
        out_specs=[
            pl.BlockSpec((_B, _DIM), lambda i: (0, 0)),
            pl.BlockSpec((1, 1), lambda i: (0, 0)),
        ],
        out_shape=[
            jax.ShapeDtypeStruct((_B, _DIM), jnp.float32),
            jax.ShapeDtypeStruct((1, 1), jnp.float32),
        ],
        scratch_shapes=[
            pltpu.VMEM((2 * _B, _EMB), jnp.float32),
            pltpu.VMEM((_K, _EMB), jnp.float32),
            pltpu.VMEM((_NE // 2, _CH * 64, _DIM), jnp.float32),
            pltpu.VMEM((9, _EMB, _EMB), jnp.float32),
            pltpu.VMEM((16 * _EMB, _EMB), jnp.float32),
            pltpu.SemaphoreType.DMA,
            pltpu.SemaphoreType.DMA((_NE // 2,)),
            pltpu.SemaphoreType.DMA((2,)),
        ],
    )(input_data_first.reshape(_B * 64, _DIM),
      input_data_last.reshape(_B * 64, _DIM), Win, b_in.reshape(1, _EMB),
      c1r, c1b.reshape(1, _EMB), c2r, c2b.reshape(1, _EMB), codebooks, rv,
      Wout, b_out.reshape(1, _DIM))
    return out.reshape(_B, 1, _DIM), perp.reshape(())


# speedup vs baseline: 1.0507x; 1.0507x over previous
import jax
import jax.numpy as jnp
from jax.experimental import pallas as pl
from jax.experimental.pallas import tpu as pltpu

_B = 64
_GRID = 8
_EMB = 256
_DIM = 1024
_K = 8192
_CH = 32
_NE = 2 * _B // _CH
_KB = _K // _NE


def _enc_body(x, w_ref, b_ref, c1_ref, c1b_ref, c2_ref, c2b_ref):
    y = jnp.dot(x, w_ref[...],
                preferred_element_type=jnp.float32) + b_ref[...]
    y4 = y.reshape(_CH, _GRID, _GRID, _EMB)
    yp = jnp.pad(y4, ((0, 0), (1, 1), (1, 1), (0, 0)))
    y6 = yp.reshape(_CH, 5, 2, 5, 2, _EMB)
    acc = jnp.broadcast_to(c1b_ref[...], (_CH * 16, _EMB))
    for di in range(3):
        fi, pi = di // 2, di % 2
        for dj in range(3):
            fj, pj = dj // 2, dj % 2
            xs = y6[:, fi:fi + 4, pi, fj:fj + 4, pj, :].reshape(
                _CH * 16, _EMB)
            acc = acc + jnp.dot(xs, c1_ref[di * 3 + dj],
                                preferred_element_type=jnp.float32)
    h = jnp.maximum(acc, 0.0).reshape(_CH, 16 * _EMB)
    return jnp.dot(h, c2_ref[...],
                   preferred_element_type=jnp.float32) + c2b_ref[...]


def _fused_kernel(x1_ref, x2_ref, w_ref, b_ref, c1_ref, c1b_ref, c2_ref,
                  c2b_ref, cbh_ref, rv_ref, wout_ref, bout_ref,
                  out_ref, perp_ref, e_ref, cb_ref, x2_sc, sem, xsem):
    i = pl.program_id(0)
    nh = _NE // 2

    @pl.when(i < _NE)
    def _():
        pltpu.make_async_copy(
            cbh_ref.at[pl.ds(i * _KB, _KB), :],
            cb_ref.at[pl.ds(i * _KB, _KB), :], sem).start()

        @pl.when(i < nh)
        def _():
            pltpu.make_async_copy(
                x2_ref.at[pl.ds(i * _CH * 64, _CH * 64), :],
                x2_sc.at[i], xsem.at[i]).start()
            e_ref[pl.ds(i * _CH, _CH), :] = _enc_body(
                x1_ref[...], w_ref, b_ref, c1_ref, c1b_ref, c2_ref, c2b_ref)

        @pl.when(i >= nh)
        def _():
            j = i - nh
            pltpu.make_async_copy(
                x2_ref.at[pl.ds(j * _CH * 64, _CH * 64), :],
                x2_sc.at[j], xsem.at[j]).wait()
            e_ref[pl.ds(i * _CH, _CH), :] = _enc_body(
                x2_sc[j], w_ref, b_ref, c1_ref, c1b_ref, c2_ref, c2b_ref)

    @pl.when(i == _NE)
    def _():
        for d in range(_NE):
            pltpu.make_async_copy(
                cbh_ref.at[pl.ds(d * _KB, _KB), :],
                cb_ref.at[pl.ds(d * _KB, _KB), :], sem).wait()
        e = e_ref[...]
        z = e[_B:, :] - e[:_B, :]
        cb = cb_ref[...]
        ones = jnp.ones((1, _EMB), dtype=jnp.float32)
        cbn = jax.lax.dot_general(ones, cb * cb, (((1,), (1,)), ((), ())),
                                  preferred_element_type=jnp.float32)
        cross = jax.lax.dot_general(z, cb, (((1,), (1,)), ((), ())),
                                    preferred_element_type=jnp.float32)
        s = cbn - 2.0 * cross
        smin = jnp.min(s, axis=1, keepdims=True)
        iota = jax.lax.broadcasted_iota(jnp.int32, (_B, _K), 1)
        idx = jnp.min(jnp.where(s <= smin, iota, _K), axis=1, keepdims=True)
        onehot = (iota == idx).astype(jnp.float32)
        hard = jnp.dot(onehot, cb, preferred_element_type=jnp.float32)

        dz = z - hard
        nq = jnp.sqrt(jnp.sum(dz * dz, axis=1, keepdims=True))
        rv = rv_ref[...]
        nr = jnp.sqrt(jnp.sum(rv * rv, axis=1, keepdims=True))
        quantized = z + (nq / (nr + 1e-12)) * rv

        ieq = (idx == jnp.transpose(idx)).astype(jnp.float32)
        counts = jnp.sum(ieq, axis=1, keepdims=True)
        lp = jnp.log(counts * (1.0 / _B) + 1e-10)
        perp_ref[...] = jnp.broadcast_to(
            jnp.exp(-jnp.sum(lp) * (1.0 / _B)), (1, 1))

        out_ref[...] = (jnp.dot(quantized, wout_ref[...],
                                preferred_element_type=jnp.float32)
                        + bout_ref[...])


def kernel(input_data_first, input_data_last, codebooks, Win, b_in, Wout,
           b_out, c1w, c1b, c2w, c2b):
    c1r = jnp.transpose(c1w, (2, 3, 1, 0)).reshape(9, _EMB, _EMB)
    c2r = jnp.transpose(c2w, (2, 3, 1, 0)).reshape(16 * _EMB, _EMB)
    rv = jax.random.normal(jax.random.key(42), (_B, _EMB), dtype=jnp.float32)

    out, perp = pl.pallas_call(
        _fused_kernel,
        grid=(_NE + 1,),
        in_specs=[
            pl.BlockSpec((_CH * 64, _DIM), lambda i: (jnp.minimum(i, _NE // 2 - 1), 0)),
            pl.BlockSpec(memory_space=pltpu.MemorySpace.HBM),
            pl.BlockSpec((_DIM, _EMB), lambda i: (0, 0)),
            pl.BlockSpec((1, _EMB), lambda i: (0, 0)),
            pl.BlockSpec((9, _EMB, _EMB), lambda i: (0, 0, 0)),
            pl.BlockSpec((1, _EMB), lambda i: (0, 0)),
            pl.BlockSpec((16 * _EMB, _EMB), lambda i: (0, 0)),
            pl.BlockSpec((1, _EMB), lambda i: (0, 0)),
            pl.BlockSpec(memory_space=pltpu.MemorySpace.HBM),
            pl.BlockSpec((_B, _EMB), lambda i: (0, 0)),
            pl.BlockSpec((_EMB, _DIM), lambda i: (0, 0)),
            pl.BlockSpec((1, _DIM), lambda i: (0, 0)),
        ],
        out_specs=[
            pl.BlockSpec((_B, _DIM), lambda i: (0, 0)),
            pl.BlockSpec((1, 1), lambda i: (0, 0)),
        ],
        out_shape=[
            jax.ShapeDtypeStruct((_B, _DIM), jnp.float32),
            jax.ShapeDtypeStruct((1, 1), jnp.float32),
        ],
        scratch_shapes=[
            pltpu.VMEM((2 * _B, _EMB), jnp.float32),
            pltpu.VMEM((_K, _EMB), jnp.float32),
            pltpu.VMEM((_NE // 2, _CH * 64, _DIM), jnp.float32),
            pltpu.SemaphoreType.DMA,
            pltpu.SemaphoreType.DMA((_NE // 2,)),
        ],
    )(input_data_first.reshape(_B * 64, _DIM),
      input_data_last.reshape(_B * 64, _DIM), Win, b_in.reshape(1, _EMB),
      c1r, c1b.reshape(1, _EMB), c2r, c2b.reshape(1, _EMB), codebooks, rv,
      Wout, b_out.reshape(1, _DIM))
    return out.reshape(_B, 1, _DIM), perp.reshape(())


# speedup vs baseline: 1.0514x; 1.0007x over previous
import jax
import jax.numpy as jnp
from jax.experimental import pallas as pl
from jax.experimental.pallas import tpu as pltpu

_B = 64
_GRID = 8
_EMB = 256
_DIM = 1024
_K = 8192
_CH = 32
_NE = 2 * _B // _CH
_KB = _K // _NE


def _enc_body(x, w_ref, b_ref, c1_ref, c1b_ref, c2_ref, c2b_ref):
    y = jnp.dot(x, w_ref[...],
                preferred_element_type=jnp.float32) + b_ref[...]
    y4 = y.reshape(_CH, _GRID, _GRID, _EMB)
    yp = jnp.pad(y4, ((0, 0), (1, 1), (1, 1), (0, 0)))
    y6 = yp.reshape(_CH, 5, 2, 5, 2, _EMB)
    acc = jnp.broadcast_to(c1b_ref[...], (_CH * 16, _EMB))
    for di in range(3):
        fi, pi = di // 2, di % 2
        for dj in range(3):
            fj, pj = dj // 2, dj % 2
            xs = y6[:, fi:fi + 4, pi, fj:fj + 4, pj, :].reshape(
                _CH * 16, _EMB)
            acc = acc + jnp.dot(xs, c1_ref[di * 3 + dj],
                                preferred_element_type=jnp.float32)
    h = jnp.maximum(acc, 0.0).reshape(_CH, 16 * _EMB)
    return jnp.dot(h, c2_ref[...],
                   preferred_element_type=jnp.float32) + c2b_ref[...]


def _fused_kernel(x1_ref, x2_ref, w_ref, b_ref, c1_ref, c1b_ref, c2_ref,
                  c2b_ref, cbh_ref, rv_ref, wout_ref, bout_ref,
                  out_ref, perp_ref, e_ref, cb_ref, x2_sc, s_ref, sem, xsem):
    i = pl.program_id(0)
    nh = _NE // 2

    @pl.when(i < _NE)
    def _():
        pltpu.make_async_copy(
            cbh_ref.at[pl.ds(i * _KB, _KB), :],
            cb_ref.at[pl.ds(i * _KB, _KB), :], sem).start()

        @pl.when(i < nh)
        def _():
            pltpu.make_async_copy(
                x2_ref.at[pl.ds(i * _CH * 64, _CH * 64), :],
                x2_sc.at[i], xsem.at[i]).start()
            e_ref[pl.ds(i * _CH, _CH), :] = _enc_body(
                x1_ref[...], w_ref, b_ref, c1_ref, c1b_ref, c2_ref, c2b_ref)

        @pl.when(i >= nh)
        def _():
            j = i - nh
            pltpu.make_async_copy(
                x2_ref.at[pl.ds(j * _CH * 64, _CH * 64), :],
                x2_sc.at[j], xsem.at[j]).wait()
            e_ref[pl.ds(i * _CH, _CH), :] = _enc_body(
                x2_sc[j], w_ref, b_ref, c1_ref, c1b_ref, c2_ref, c2b_ref)

    @pl.when(i == _NE)
    def _():
        for d in range(_NE):
            pltpu.make_async_copy(
                cbh_ref.at[pl.ds(d * _KB, _KB), :],
                cb_ref.at[pl.ds(d * _KB, _KB), :], sem).wait()
        e = e_ref[...]
        z = e[_B:, :] - e[:_B, :]
        cb = cb_ref[...]
        ones = jnp.ones((1, _EMB), dtype=jnp.float32)
        cbn = jax.lax.dot_general(ones, cb * cb, (((1,), (1,)), ((), ())),
                                  preferred_element_type=jnp.float32)
        cross = jax.lax.dot_general(z, cb, (((1,), (1,)), ((), ())),
                                    preferred_element_type=jnp.float32)
        s_ref[...] = cbn - 2.0 * cross
        s = s_ref[...]
        smin = jnp.min(s, axis=1, keepdims=True)
        iota = jax.lax.broadcasted_iota(jnp.int32, (_B, _K), 1)
        idx = jnp.min(jnp.where(s <= smin, iota, _K), axis=1, keepdims=True)
        onehot = (iota == idx).astype(jnp.float32)
        hard = jnp.dot(onehot, cb, preferred_element_type=jnp.float32)

        dz = z - hard
        nq = jnp.sqrt(jnp.sum(dz * dz, axis=1, keepdims=True))
        rv = rv_ref[...]
        nr = jnp.sqrt(jnp.sum(rv * rv, axis=1, keepdims=True))
        quantized = z + (nq / (nr + 1e-12)) * rv

        ieq = (idx == jnp.transpose(idx)).astype(jnp.float32)
        counts = jnp.sum(ieq, axis=1, keepdims=True)
        lp = jnp.log(counts * (1.0 / _B) + 1e-10)
        perp_ref[...] = jnp.broadcast_to(
            jnp.exp(-jnp.sum(lp) * (1.0 / _B)), (1, 1))

        out_ref[...] = (jnp.dot(quantized, wout_ref[...],
                                preferred_element_type=jnp.float32)
                        + bout_ref[...])


def kernel(input_data_first, input_data_last, codebooks, Win, b_in, Wout,
           b_out, c1w, c1b, c2w, c2b):
    c1r = jnp.transpose(c1w, (2, 3, 1, 0)).reshape(9, _EMB, _EMB)
    c2r = jnp.transpose(c2w, (2, 3, 1, 0)).reshape(16 * _EMB, _EMB)
    rv = jax.random.normal(jax.random.key(42), (_B, _EMB), dtype=jnp.float32)

    out, perp = pl.pallas_call(
        _fused_kernel,
        grid=(_NE + 1,),
        in_specs=[
            pl.BlockSpec((_CH * 64, _DIM), lambda i: (jnp.minimum(i, _NE // 2 - 1), 0)),
            pl.BlockSpec(memory_space=pltpu.MemorySpace.HBM),
            pl.BlockSpec((_DIM, _EMB), lambda i: (0, 0)),
            pl.BlockSpec((1, _EMB), lambda i: (0, 0)),
            pl.BlockSpec((9, _EMB, _EMB), lambda i: (0, 0, 0)),
            pl.BlockSpec((1, _EMB), lambda i: (0, 0)),
            pl.BlockSpec((16 * _EMB, _EMB), lambda i: (0, 0)),
            pl.BlockSpec((1, _EMB), lambda i: (0, 0)),
            pl.BlockSpec(memory_space=pltpu.MemorySpace.HBM),
            pl.BlockSpec((_B, _EMB), lambda i: (0, 0)),
            pl.BlockSpec((_EMB, _DIM), lambda i: (0, 0)),
            pl.BlockSpec((1, _DIM), lambda i: (0, 0)),
        ],
        out_specs=[
            pl.BlockSpec((_B, _DIM), lambda i: (0, 0)),
            pl.BlockSpec((1, 1), lambda i: (0, 0)),
        ],
        out_shape=[
            jax.ShapeDtypeStruct((_B, _DIM), jnp.float32),
            jax.ShapeDtypeStruct((1, 1), jnp.float32),
        ],
        scratch_shapes=[
            pltpu.VMEM((2 * _B, _EMB), jnp.float32),
            pltpu.VMEM((_K, _EMB), jnp.float32),
            pltpu.VMEM((_NE // 2, _CH * 64, _DIM), jnp.float32),
            pltpu.VMEM((_B, _K), jnp.float32),
            pltpu.SemaphoreType.DMA,
            pltpu.SemaphoreType.DMA((_NE // 2,)),
        ],
    )(input_data_first.reshape(_B * 64, _DIM),
      input_data_last.reshape(_B * 64, _DIM), Win, b_in.reshape(1, _EMB),
      c1r, c1b.reshape(1, _EMB), c2r, c2b.reshape(1, _EMB), codebooks, rv,
      Wout, b_out.reshape(1, _DIM))
    return out.reshape(_B, 1, _DIM), perp.reshape(())
